# baseline (speedup 1.0000x reference)
import functools

import jax
import jax.numpy as jnp
from jax import lax
from jax.experimental import pallas as pl
from jax.experimental.pallas import tpu as pltpu
from jax.experimental.pallas import tpu_sc as plsc

B = 512
D = 768
H1 = 128
CPW = 4
_NEG = -3e38
_POS = 3e38


def _shr0(a, k, fill):
    f = jnp.full((k,) + a.shape[1:], fill, a.dtype)
    return jnp.concatenate([f, a[:-k]], axis=0)


def _shl0(a, k, fill):
    f = jnp.full((k,) + a.shape[1:], fill, a.dtype)
    return jnp.concatenate([a[k:], f], axis=0)


def _sort_kernel(w1_ref, b1_ref, x_ref, m_ref, ix_ref):
    h = lax.dot_general(x_ref[...], w1_ref[...], (((1,), (1,)), ((), ())),
                        preferred_element_type=jnp.float32)
    v = jnp.maximum(h + b1_ref[...], 0.0)

    ix = lax.broadcasted_iota(jnp.int32, (B, H1), 0)
    pidx = lax.broadcasted_iota(jnp.int32, (B, 1), 0)
    k = 2
    while k <= B:
        j = k // 2
        while j >= 1:
            mj = (pidx & j) != 0
            sm = ((pidx & k) == 0) != mj
            pv = jnp.where(mj, _shr0(v, j, 0.0), _shl0(v, j, 0.0))
            pi = jnp.where(mj, _shr0(ix, j, 0), _shl0(ix, j, 0))
            nv = jnp.where(sm, jnp.minimum(v, pv), jnp.maximum(v, pv))
            ix = jnp.where(nv == v, ix, pi)
            v = nv
            j //= 2
        k *= 2

    s = [_shr0(v, t, _NEG) for t in range(5, 0, -1)] + [v] + \
        [_shl0(v, t, _POS) for t in range(1, 6)]
    z = [_shr0(v, t, 0.0) for t in range(5, 0, -1)] + [v] + \
        [_shl0(v, t, 0.0) for t in range(1, 6)]
    x0 = v
    wsum = z[5] + z[6] + z[7] + z[8] + z[9] + z[10]
    best_cost = jnp.maximum(x0 - s[5], s[10] - x0)
    best_sum = wsum
    for t in range(1, 6):
        wsum = wsum + z[5 - t] - z[11 - t]
        cost = jnp.maximum(x0 - s[5 - t], s[10 - t] - x0)
        take = cost < best_cost
        best_cost = jnp.where(take, cost, best_cost)
        best_sum = jnp.where(take, wsum, best_sum)
    m = best_sum * jnp.float32(1.0 / 6.0)

    m_ref[...] = m.T
    ix_ref[...] = ix.T


def _sc_scatter(m_hbm, ix_hbm, out_hbm, mv, iv, ov, sem):
    info = plsc.get_sparse_core_info()
    nc = info.num_cores
    wid = lax.axis_index("s") * nc + lax.axis_index("c")
    f0 = wid * CPW

    copies = []
    for c in range(CPW):
        copies.append(
            pltpu.async_copy(m_hbm.at[f0 + c], mv.at[pl.ds(c * B, B)], sem))
        copies.append(
            pltpu.async_copy(ix_hbm.at[f0 + c], iv.at[pl.ds(c * B, B)], sem))
    for cp in copies:
        cp.wait()

    def scat_body(kk, _):
        c = kk // 32
        idx = iv[pl.ds(kk * 16, 16)]
        val = mv[pl.ds(kk * 16, 16)]
        plsc.store_scatter(ov, [idx + c * B], val)
        return 0
    lax.fori_loop(0, CPW * 32, scat_body, 0, unroll=8)

    copies = [pltpu.async_copy(ov.at[pl.ds(c * B, B)], out_hbm.at[f0 + c], sem)
              for c in range(CPW)]
    for cp in copies:
        cp.wait()


def _sc_scatter_call(m_T, ix_T):
    call = pl.kernel(
        _sc_scatter,
        mesh=plsc.VectorSubcoreMesh(core_axis_name="c", subcore_axis_name="s"),
        out_type=jax.ShapeDtypeStruct((H1, B), jnp.float32),
        scratch_types=[
            pltpu.VMEM((CPW * B,), jnp.float32),
            pltpu.VMEM((CPW * B,), jnp.int32),
            pltpu.VMEM((CPW * B,), jnp.float32),
            pltpu.SemaphoreType.DMA,
        ],
        compiler_params=pltpu.CompilerParams(needs_layout_passes=False,
                                             skip_device_barrier=True),
    )
    return call(m_T, ix_T)


def _dense_kernel(t_ref, wpr_ref, bpr_ref, w2_ref, b2_ref, w3_ref, b3_ref,
                  out_ref):
    t = t_ref[...]
    pr = lax.dot_general(wpr_ref[...], t, (((1,), (0,)), ((), ())),
                         preferred_element_type=jnp.float32)
    pr = jnp.maximum(pr + bpr_ref[...], 0.0)
    h2 = lax.dot_general(w2_ref[...], pr, (((1,), (0,)), ((), ())),
                         preferred_element_type=jnp.float32)
    h2 = jnp.maximum(h2 + b2_ref[...], 0.0)
    out = lax.dot_general(h2, w3_ref[...], (((0,), (1,)), ((), ())),
                          preferred_element_type=jnp.float32)
    out_ref[...] = out + b3_ref[...]


@jax.jit
def kernel(input, W1, b1, Wpr, bpr, W2, b2, W3, b3):
    m_T, ix_T = pl.pallas_call(
        _sort_kernel,
        out_shape=[
            jax.ShapeDtypeStruct((H1, B), jnp.float32),
            jax.ShapeDtypeStruct((H1, B), jnp.int32),
        ],
    )(W1, b1.reshape(1, H1), input)

    t_T = _sc_scatter_call(m_T, ix_T)

    out = pl.pallas_call(
        _dense_kernel,
        out_shape=jax.ShapeDtypeStruct((B, W3.shape[0]), jnp.float32),
    )(t_T, Wpr, bpr.reshape(-1, 1), W2, b2.reshape(-1, 1), W3,
      b3.reshape(1, -1))
    return out
